# baseline (speedup 1.0000x reference)
import functools

import jax
import jax.numpy as jnp
from jax import lax
from jax.experimental import pallas as pl
from jax.experimental.pallas import tpu as pltpu
from jax.experimental.pallas import tpu_sc as plsc

B, N, K = 4, 1024, 20
KP = 20
BN = B * N
NEDGE = BN * K
EPS = 1e-5
BLK = 256
TP = 128
NW = 32
PW = BN // NW
PB = 128
NBLK = BN // PB


def _knn_body(xt_ref, pts_ref, out_ref):
    b = pl.program_id(0)
    xt = xt_ref[0]
    pi = pts_ref[0]
    x0, x1, x2 = xt[0:1, :], xt[1:2, :], xt[2:3, :]
    p0, p1, p2 = pi[:, 0:1], pi[:, 1:2], pi[:, 2:3]
    g = jnp.dot(pi.astype(jnp.bfloat16), xt.astype(jnp.bfloat16),
                preferred_element_type=jnp.float32)
    sqj = (x0 * x0 + x1 * x1) + x2 * x2
    sqi = (p0 * p0 + p1 * p1) + p2 * p2
    d = (sqi + sqj) - 2.0 * g
    iota = lax.broadcasted_iota(jnp.int32, (BLK, N), 1)
    cols = []
    for _ in range(K):
        m = jnp.min(d, axis=1, keepdims=True)
        cand = jnp.where(d <= m, iota, N)
        amin = jnp.min(cand, axis=1, keepdims=True)
        cols.append(amin)
        d = jnp.where(iota == amin, jnp.float32(jnp.inf), d)
    cols += [cols[0]] * (KP - K)
    out_ref[...] = jnp.concatenate(cols, axis=1) + b * N


def _knn(x):
    pts = jnp.transpose(x, (0, 2, 1))
    return pl.pallas_call(
        _knn_body,
        grid=(B, N // BLK),
        in_specs=[
            pl.BlockSpec((1, 3, N), lambda b, i: (b, 0, 0)),
            pl.BlockSpec((1, BLK, 3), lambda b, i: (b, i, 0)),
        ],
        out_specs=pl.BlockSpec((BLK, KP), lambda b, i: (b * (N // BLK) + i, 0)),
        out_shape=jax.ShapeDtypeStruct((BN, KP), jnp.int32),
    )(x, pts)


CH = 4
NCH = PW // CH
CR = CH * KP


def _make_gather():
    mesh = plsc.VectorSubcoreMesh(core_axis_name="c", subcore_axis_name="s")

    @functools.partial(
        pl.kernel,
        mesh=mesh,
        out_type=jax.ShapeDtypeStruct((BN * KP, TP), jnp.float32),
        scratch_types=[
            pltpu.VMEM((PW * KP,), jnp.int32),
            pltpu.VMEM((CR, TP), jnp.float32),
            pltpu.VMEM((CR, TP), jnp.float32),
            pltpu.SemaphoreType.DMA,
            pltpu.SemaphoreType.DMA,
            pltpu.SemaphoreType.DMA,
            pltpu.SemaphoreType.DMA,
        ],
    )
    def gather(x_hbm, idx_hbm, g_hbm, idx_v, rows0, rows1,
               sg0, sg1, sw0, sw1):
        wid = lax.axis_index("s") * 2 + lax.axis_index("c")
        base = wid * PW
        pltpu.sync_copy(idx_hbm.at[pl.ds(base * KP, PW * KP)], idx_v)

        def isl(c):
            return idx_v.at[pl.ds(c * CR, CR)]

        def gsl(c):
            return g_hbm.at[pl.ds(base * KP + c * CR, CR)]

        pltpu.async_copy(x_hbm.at[isl(0)], rows0, sg0)

        def body(i, carry):
            c0 = 2 * i
            c1 = c0 + 1
            pltpu.make_async_copy(x_hbm.at[isl(c0)], rows0, sg0).wait()

            @pl.when(i >= 1)
            def _():
                pltpu.make_async_copy(rows1, gsl(c1 - 2), sw1).wait()

            hg1 = pltpu.async_copy(x_hbm.at[isl(c1)], rows1, sg1)
            hw0 = pltpu.async_copy(rows0, gsl(c0), sw0)
            hg1.wait()
            hw0.wait()

            @pl.when(i < NCH // 2 - 1)
            def _():
                pltpu.async_copy(x_hbm.at[isl(c0 + 2)], rows0, sg0)

            pltpu.async_copy(rows1, gsl(c1), sw1)
            return carry

        lax.fori_loop(0, NCH // 2, body, 0)
        pltpu.make_async_copy(rows1, gsl(NCH - 1), sw1).wait()

    return gather


def _edge_body(g_ref, x_ref, zc_ref, wd_ref, m_ref, p1_ref, p2_ref):
    X = x_ref[...]
    Zc = zc_ref[...]
    Wd = wd_ref[...].astype(jnp.bfloat16)
    h = jnp.dot((g_ref[:, 0, :] - X).astype(jnp.bfloat16), Wd,
                preferred_element_type=jnp.float32) + Zc
    M = h
    s1 = h
    s2 = h * h
    for k in range(1, K):
        h = jnp.dot((g_ref[:, k, :] - X).astype(jnp.bfloat16), Wd,
                    preferred_element_type=jnp.float32) + Zc
        M = jnp.maximum(M, h)
        s1 = s1 + h
        s2 = s2 + h * h
    m_ref[...] = M
    p1_ref[...] = jnp.sum(s1, axis=0, keepdims=True)[None]
    p2_ref[...] = jnp.sum(s2, axis=0, keepdims=True)[None]


def _edge(G3, table, Zc, WdT, Cn):
    return pl.pallas_call(
        _edge_body,
        grid=(NBLK,),
        in_specs=[
            pl.BlockSpec((PB, KP, TP), lambda i: (i, 0, 0)),
            pl.BlockSpec((PB, TP), lambda i: (i, 0)),
            pl.BlockSpec((PB, Cn), lambda i: (i, 0)),
            pl.BlockSpec((TP, Cn), lambda i: (0, 0)),
        ],
        out_specs=[
            pl.BlockSpec((PB, Cn), lambda i: (i, 0)),
            pl.BlockSpec((1, 1, Cn), lambda i: (i, 0, 0)),
            pl.BlockSpec((1, 1, Cn), lambda i: (i, 0, 0)),
        ],
        out_shape=[
            jax.ShapeDtypeStruct((BN, Cn), jnp.float32),
            jax.ShapeDtypeStruct((NBLK, 1, Cn), jnp.float32),
            jax.ShapeDtypeStruct((NBLK, 1, Cn), jnp.float32),
        ],
    )(G3, table, Zc, WdT)


def _bn_stats(p1, p2, g, b):
    s1 = jnp.sum(p1.reshape(NBLK, -1), axis=0, keepdims=True)
    s2 = jnp.sum(p2.reshape(NBLK, -1), axis=0, keepdims=True)
    mean = s1 * (1.0 / NEDGE)
    var = s2 * (1.0 / NEDGE) - mean * mean
    scale = g / jnp.sqrt(var + EPS)
    shift = b - mean * scale
    return scale, shift


def _bn_mid_body(m_ref, p1_ref, p2_ref, g_ref, b_ref, wc_ref,
                 o_ref, zc_ref):
    scale, shift = _bn_stats(p1_ref[...], p2_ref[...], g_ref[...], b_ref[...])
    o = jnp.maximum(m_ref[...] * scale + shift, 0.0)
    o_ref[...] = o
    zc_ref[...] = jnp.dot(o.astype(jnp.bfloat16),
                          wc_ref[...].astype(jnp.bfloat16),
                          preferred_element_type=jnp.float32)


def _bn_mid_pad_body(m_ref, p1_ref, p2_ref, g_ref, b_ref, wc_ref,
                     o_ref, zc_ref, tab_ref):
    scale, shift = _bn_stats(p1_ref[...], p2_ref[...], g_ref[...], b_ref[...])
    o = jnp.maximum(m_ref[...] * scale + shift, 0.0)
    o_ref[...] = o
    zc_ref[...] = jnp.dot(o.astype(jnp.bfloat16),
                          wc_ref[...].astype(jnp.bfloat16),
                          preferred_element_type=jnp.float32)
    tab_ref[...] = jnp.concatenate(
        [o, jnp.zeros((BN, TP - o.shape[1]), jnp.float32)], axis=1)


def _bn_last_body(m_ref, p1_ref, p2_ref, g_ref, b_ref, o_ref):
    scale, shift = _bn_stats(p1_ref[...], p2_ref[...], g_ref[...], b_ref[...])
    o_ref[...] = jnp.maximum(m_ref[...] * scale + shift, 0.0)


def _bn_mid(M, P1, P2, g, b, WcT, Cn2, pad_table):
    C = M.shape[1]
    outs = [jax.ShapeDtypeStruct((BN, C), jnp.float32),
            jax.ShapeDtypeStruct((BN, Cn2), jnp.float32)]
    body = _bn_mid_body
    if pad_table:
        outs.append(jax.ShapeDtypeStruct((BN, TP), jnp.float32))
        body = _bn_mid_pad_body
    return pl.pallas_call(
        body,
        out_shape=outs,
    )(M, P1, P2, g.reshape(1, C), b.reshape(1, C), WcT)


def _bn_last(M, P1, P2, g, b):
    C = M.shape[1]
    return pl.pallas_call(
        _bn_last_body,
        out_shape=jax.ShapeDtypeStruct((BN, C), jnp.float32),
    )(M, P1, P2, g.reshape(1, C), b.reshape(1, C))


def _prep_body(x_ref, w_ref, zc_ref):
    zc_ref[...] = jnp.dot(x_ref[...].astype(jnp.bfloat16),
                          w_ref[...].astype(jnp.bfloat16),
                          preferred_element_type=jnp.float32)


def _prep(X0p, Wc0T):
    return pl.pallas_call(
        _prep_body,
        out_shape=jax.ShapeDtypeStruct((BN, 64), jnp.float32),
    )(X0p, Wc0T)


def _final_body(o0_ref, o1_ref, o2_ref, w0_ref, w1_ref, w2_ref, g_ref, b_ref,
                out_ref):
    h = (jnp.dot(o0_ref[...].astype(jnp.bfloat16),
                 w0_ref[...].astype(jnp.bfloat16),
                 preferred_element_type=jnp.float32)
         + jnp.dot(o1_ref[...].astype(jnp.bfloat16),
                   w1_ref[...].astype(jnp.bfloat16),
                   preferred_element_type=jnp.float32)
         + jnp.dot(o2_ref[...].astype(jnp.bfloat16),
                   w2_ref[...].astype(jnp.bfloat16),
                   preferred_element_type=jnp.float32))
    mean = jnp.mean(h, axis=0, keepdims=True)
    var = jnp.mean((h - mean) * (h - mean), axis=0, keepdims=True)
    scale = g_ref[...] / jnp.sqrt(var + EPS)
    shift = b_ref[...] - mean * scale
    out_ref[...] = jnp.maximum(h * scale + shift, 0.0)


def _final(o0, o1, o2, WfT, gf, bf):
    return pl.pallas_call(
        _final_body,
        out_shape=jax.ShapeDtypeStruct((BN, 512), jnp.float32),
    )(o0, o1, o2, WfT[:64], WfT[64:192], WfT[192:448],
      gf.reshape(1, 512), bf.reshape(1, 512))


def _pad_rows(W, rows):
    return jnp.pad(W, ((0, rows - W.shape[0]), (0, 0)))


def kernel(x, W0, g0, b0, W1, g1, b1, W2, g2, b2, Wf, gf, bf):
    idxg = _knn(x).reshape(BN * KP)

    Xr = jnp.transpose(x, (0, 2, 1)).reshape(BN, 3)
    X0p = jnp.pad(Xr, ((0, 0), (0, TP - 3)))

    gth = _make_gather()

    Wd0T = _pad_rows(W0[:, :3].T, TP)
    Wc0T = _pad_rows(W0[:, 3:].T, TP)
    Zc0 = _prep(X0p, Wc0T)
    G0 = gth(X0p, idxg).reshape(BN, KP, TP)
    M0, P10, P20 = _edge(G0, X0p, Zc0, Wd0T, 64)
    o0, Zc1, tab1 = _bn_mid(M0, P10, P20, g0, b0, W1[:, 64:].T, 128,
                            pad_table=True)

    Wd1T = _pad_rows(W1[:, :64].T, TP)
    G1 = gth(tab1, idxg).reshape(BN, KP, TP)
    M1, P11, P21 = _edge(G1, tab1, Zc1, Wd1T, 128)
    o1, Zc2 = _bn_mid(M1, P11, P21, g1, b1, W2[:, 128:].T, 256,
                      pad_table=False)

    Wd2T = W2[:, :128].T
    G2 = gth(o1, idxg).reshape(BN, KP, TP)
    M2, P12, P22 = _edge(G2, o1, Zc2, Wd2T, 256)
    o2 = _bn_last(M2, P12, P22, g2, b2)

    out = _final(o0, o1, o2, Wf.T, gf, bf)
    return out.reshape(B, N, 512)


# speedup vs baseline: 1.3861x; 1.3861x over previous
import functools

import jax
import jax.numpy as jnp
from jax import lax
from jax.experimental import pallas as pl
from jax.experimental.pallas import tpu as pltpu
from jax.experimental.pallas import tpu_sc as plsc

B, N, K = 4, 1024, 20
KP = 24
BN = B * N
HN = BN // 2
HB = B // 2
NEDGE = BN * K
EPS = 1e-5
BLK = 256
TP = 128
NW = 32
PW = HN // NW
PB = 128
NBLK = HN // PB


def _knn_body(b0, xt_ref, pts_ref, out_ref):
    b = pl.program_id(0)
    xt = xt_ref[0]
    pi = pts_ref[0]
    x0, x1, x2 = xt[0:1, :], xt[1:2, :], xt[2:3, :]
    p0, p1, p2 = pi[:, 0:1], pi[:, 1:2], pi[:, 2:3]
    g = jnp.dot(pi.astype(jnp.bfloat16), xt.astype(jnp.bfloat16),
                preferred_element_type=jnp.float32)
    sqj = (x0 * x0 + x1 * x1) + x2 * x2
    sqi = (p0 * p0 + p1 * p1) + p2 * p2
    d = (sqi + sqj) - 2.0 * g
    iota = lax.broadcasted_iota(jnp.int32, (BLK, N), 1)
    cols = []
    for _ in range(K):
        m = jnp.min(d, axis=1, keepdims=True)
        cand = jnp.where(d <= m, iota, N)
        amin = jnp.min(cand, axis=1, keepdims=True)
        cols.append(amin)
        d = jnp.where(iota == amin, jnp.float32(jnp.inf), d)
    cols += [cols[0]] * (KP - K)
    out_ref[...] = jnp.concatenate(cols, axis=1) + (b0 + b) * N


def _knn(x, b0):
    pts = jnp.transpose(x, (0, 2, 1))
    return pl.pallas_call(
        functools.partial(_knn_body, b0),
        grid=(HB, N // BLK),
        in_specs=[
            pl.BlockSpec((1, 3, N), lambda b, i: (b0 + b, 0, 0)),
            pl.BlockSpec((1, BLK, 3), lambda b, i: (b0 + b, i, 0)),
        ],
        out_specs=pl.BlockSpec((BLK, KP), lambda b, i: (b * (N // BLK) + i, 0)),
        out_shape=jax.ShapeDtypeStruct((HN, KP), jnp.int32),
    )(x, pts)


CH = 4
NCH = PW // CH
CR = CH * KP


def _make_gather():
    mesh = plsc.VectorSubcoreMesh(core_axis_name="c", subcore_axis_name="s")

    @functools.partial(
        pl.kernel,
        mesh=mesh,
        out_type=jax.ShapeDtypeStruct((HN * KP, TP), jnp.float32),
        scratch_types=[
            pltpu.VMEM((PW * KP,), jnp.int32),
            pltpu.VMEM((CR, TP), jnp.float32),
            pltpu.VMEM((CR, TP), jnp.float32),
            pltpu.SemaphoreType.DMA,
            pltpu.SemaphoreType.DMA,
            pltpu.SemaphoreType.DMA,
            pltpu.SemaphoreType.DMA,
        ],
    )
    def gather(x_hbm, idx_hbm, g_hbm, idx_v, rows0, rows1,
               sg0, sg1, sw0, sw1):
        wid = lax.axis_index("s") * 2 + lax.axis_index("c")
        base = wid * PW
        pltpu.sync_copy(idx_hbm.at[pl.ds(base * KP, PW * KP)], idx_v)

        def isl(c):
            return idx_v.at[pl.ds(c * CR, CR)]

        def gsl(c):
            return g_hbm.at[pl.ds(base * KP + c * CR, CR)]

        pltpu.async_copy(x_hbm.at[isl(0)], rows0, sg0)

        def body(i, carry):
            c0 = 2 * i
            c1 = c0 + 1
            pltpu.make_async_copy(x_hbm.at[isl(c0)], rows0, sg0).wait()

            @pl.when(i >= 1)
            def _():
                pltpu.make_async_copy(rows1, gsl(c1 - 2), sw1).wait()

            hg1 = pltpu.async_copy(x_hbm.at[isl(c1)], rows1, sg1)
            hw0 = pltpu.async_copy(rows0, gsl(c0), sw0)
            hg1.wait()
            hw0.wait()

            @pl.when(i < NCH // 2 - 1)
            def _():
                pltpu.async_copy(x_hbm.at[isl(c0 + 2)], rows0, sg0)

            pltpu.async_copy(rows1, gsl(c1), sw1)
            return carry

        lax.fori_loop(0, NCH // 2, body, 0)
        pltpu.make_async_copy(rows1, gsl(NCH - 1), sw1).wait()

    return gather


def _edge_body(g_ref, x_ref, zc_ref, wd_ref, m_ref, p1_ref, p2_ref):
    X = x_ref[...]
    Zc = zc_ref[...]
    Wd = wd_ref[...].astype(jnp.bfloat16)
    h = jnp.dot((g_ref[:, 0, :] - X).astype(jnp.bfloat16), Wd,
                preferred_element_type=jnp.float32) + Zc
    M = h
    s1 = h
    s2 = h * h
    for k in range(1, K):
        h = jnp.dot((g_ref[:, k, :] - X).astype(jnp.bfloat16), Wd,
                    preferred_element_type=jnp.float32) + Zc
        M = jnp.maximum(M, h)
        s1 = s1 + h
        s2 = s2 + h * h
    m_ref[...] = M
    p1_ref[...] = jnp.sum(s1, axis=0, keepdims=True)[None]
    p2_ref[...] = jnp.sum(s2, axis=0, keepdims=True)[None]


def _edge(G3, table, Zc, WdT, Cn, half):
    off = half * NBLK
    return pl.pallas_call(
        _edge_body,
        grid=(NBLK,),
        in_specs=[
            pl.BlockSpec((PB, KP, TP), lambda i: (i, 0, 0)),
            pl.BlockSpec((PB, TP), lambda i: (off + i, 0)),
            pl.BlockSpec((PB, Cn), lambda i: (off + i, 0)),
            pl.BlockSpec((TP, Cn), lambda i: (0, 0)),
        ],
        out_specs=[
            pl.BlockSpec((PB, Cn), lambda i: (i, 0)),
            pl.BlockSpec((1, 1, Cn), lambda i: (i, 0, 0)),
            pl.BlockSpec((1, 1, Cn), lambda i: (i, 0, 0)),
        ],
        out_shape=[
            jax.ShapeDtypeStruct((HN, Cn), jnp.float32),
            jax.ShapeDtypeStruct((NBLK, 1, Cn), jnp.float32),
            jax.ShapeDtypeStruct((NBLK, 1, Cn), jnp.float32),
        ],
    )(G3, table, Zc, WdT)


def _bn_stats(p1a, p1b, p2a, p2b, g, b):
    s1 = (jnp.sum(p1a.reshape(NBLK, -1), axis=0, keepdims=True)
          + jnp.sum(p1b.reshape(NBLK, -1), axis=0, keepdims=True))
    s2 = (jnp.sum(p2a.reshape(NBLK, -1), axis=0, keepdims=True)
          + jnp.sum(p2b.reshape(NBLK, -1), axis=0, keepdims=True))
    mean = s1 * (1.0 / NEDGE)
    var = s2 * (1.0 / NEDGE) - mean * mean
    scale = g / jnp.sqrt(var + EPS)
    shift = b - mean * scale
    return scale, shift


def _bn_mid_body(ma_ref, mb_ref, p1a_ref, p1b_ref, p2a_ref, p2b_ref,
                 g_ref, b_ref, wc_ref, o_ref, zc_ref):
    scale, shift = _bn_stats(p1a_ref[...], p1b_ref[...],
                             p2a_ref[...], p2b_ref[...],
                             g_ref[...], b_ref[...])
    M = jnp.concatenate([ma_ref[...], mb_ref[...]], axis=0)
    o = jnp.maximum(M * scale + shift, 0.0)
    o_ref[...] = o
    zc_ref[...] = jnp.dot(o.astype(jnp.bfloat16),
                          wc_ref[...].astype(jnp.bfloat16),
                          preferred_element_type=jnp.float32)


def _bn_mid_pad_body(ma_ref, mb_ref, p1a_ref, p1b_ref, p2a_ref, p2b_ref,
                     g_ref, b_ref, wc_ref, o_ref, zc_ref, tab_ref):
    scale, shift = _bn_stats(p1a_ref[...], p1b_ref[...],
                             p2a_ref[...], p2b_ref[...],
                             g_ref[...], b_ref[...])
    M = jnp.concatenate([ma_ref[...], mb_ref[...]], axis=0)
    o = jnp.maximum(M * scale + shift, 0.0)
    o_ref[...] = o
    zc_ref[...] = jnp.dot(o.astype(jnp.bfloat16),
                          wc_ref[...].astype(jnp.bfloat16),
                          preferred_element_type=jnp.float32)
    tab_ref[...] = jnp.concatenate(
        [o, jnp.zeros((BN, TP - o.shape[1]), jnp.float32)], axis=1)


def _bn_last_body(ma_ref, mb_ref, p1a_ref, p1b_ref, p2a_ref, p2b_ref,
                  g_ref, b_ref, o_ref):
    scale, shift = _bn_stats(p1a_ref[...], p1b_ref[...],
                             p2a_ref[...], p2b_ref[...],
                             g_ref[...], b_ref[...])
    M = jnp.concatenate([ma_ref[...], mb_ref[...]], axis=0)
    o_ref[...] = jnp.maximum(M * scale + shift, 0.0)


def _bn_mid(MA, MB, P1A, P1B, P2A, P2B, g, b, WcT, Cn2, pad_table):
    C = MA.shape[1]
    outs = [jax.ShapeDtypeStruct((BN, C), jnp.float32),
            jax.ShapeDtypeStruct((BN, Cn2), jnp.float32)]
    body = _bn_mid_body
    if pad_table:
        outs.append(jax.ShapeDtypeStruct((BN, TP), jnp.float32))
        body = _bn_mid_pad_body
    return pl.pallas_call(
        body,
        out_shape=outs,
    )(MA, MB, P1A, P1B, P2A, P2B, g.reshape(1, C), b.reshape(1, C), WcT)


def _bn_last(MA, MB, P1A, P1B, P2A, P2B, g, b):
    C = MA.shape[1]
    return pl.pallas_call(
        _bn_last_body,
        out_shape=jax.ShapeDtypeStruct((BN, C), jnp.float32),
    )(MA, MB, P1A, P1B, P2A, P2B, g.reshape(1, C), b.reshape(1, C))


def _prep_body(x_ref, w_ref, zc_ref):
    zc_ref[...] = jnp.dot(x_ref[...].astype(jnp.bfloat16),
                          w_ref[...].astype(jnp.bfloat16),
                          preferred_element_type=jnp.float32)


def _prep(X0p, Wc0T):
    return pl.pallas_call(
        _prep_body,
        out_shape=jax.ShapeDtypeStruct((BN, 64), jnp.float32),
    )(X0p, Wc0T)


def _final_body(o0_ref, o1_ref, o2_ref, w0_ref, w1_ref, w2_ref, g_ref, b_ref,
                out_ref):
    h = (jnp.dot(o0_ref[...].astype(jnp.bfloat16),
                 w0_ref[...].astype(jnp.bfloat16),
                 preferred_element_type=jnp.float32)
         + jnp.dot(o1_ref[...].astype(jnp.bfloat16),
                   w1_ref[...].astype(jnp.bfloat16),
                   preferred_element_type=jnp.float32)
         + jnp.dot(o2_ref[...].astype(jnp.bfloat16),
                   w2_ref[...].astype(jnp.bfloat16),
                   preferred_element_type=jnp.float32))
    mean = jnp.mean(h, axis=0, keepdims=True)
    var = jnp.mean((h - mean) * (h - mean), axis=0, keepdims=True)
    scale = g_ref[...] / jnp.sqrt(var + EPS)
    shift = b_ref[...] - mean * scale
    out_ref[...] = jnp.maximum(h * scale + shift, 0.0)


def _final(o0, o1, o2, WfT, gf, bf):
    return pl.pallas_call(
        _final_body,
        out_shape=jax.ShapeDtypeStruct((BN, 512), jnp.float32),
    )(o0, o1, o2, WfT[:64], WfT[64:192], WfT[192:448],
      gf.reshape(1, 512), bf.reshape(1, 512))


def _pad_rows(W, rows):
    return jnp.pad(W, ((0, rows - W.shape[0]), (0, 0)))


def kernel(x, W0, g0, b0, W1, g1, b1, W2, g2, b2, Wf, gf, bf):
    idxA = _knn(x, 0).reshape(HN * KP)
    idxB = _knn(x, HB).reshape(HN * KP)

    Xr = jnp.transpose(x, (0, 2, 1)).reshape(BN, 3)
    X0p = jnp.pad(Xr, ((0, 0), (0, TP - 3)))

    gth = _make_gather()

    Wd0T = _pad_rows(W0[:, :3].T, TP)
    Wc0T = _pad_rows(W0[:, 3:].T, TP)
    Zc0 = _prep(X0p, Wc0T)
    G0A = gth(X0p, idxA).reshape(HN, KP, TP)
    G0B = gth(X0p, idxB).reshape(HN, KP, TP)
    MA0, P1A0, P2A0 = _edge(G0A, X0p, Zc0, Wd0T, 64, 0)
    MB0, P1B0, P2B0 = _edge(G0B, X0p, Zc0, Wd0T, 64, 1)
    o0, Zc1, tab1 = _bn_mid(MA0, MB0, P1A0, P1B0, P2A0, P2B0, g0, b0,
                            W1[:, 64:].T, 128, pad_table=True)

    Wd1T = _pad_rows(W1[:, :64].T, TP)
    G1A = gth(tab1, idxA).reshape(HN, KP, TP)
    G1B = gth(tab1, idxB).reshape(HN, KP, TP)
    MA1, P1A1, P2A1 = _edge(G1A, tab1, Zc1, Wd1T, 128, 0)
    MB1, P1B1, P2B1 = _edge(G1B, tab1, Zc1, Wd1T, 128, 1)
    o1, Zc2 = _bn_mid(MA1, MB1, P1A1, P1B1, P2A1, P2B1, g1, b1,
                      W2[:, 128:].T, 256, pad_table=False)

    Wd2T = W2[:, :128].T
    G2A = gth(o1, idxA).reshape(HN, KP, TP)
    G2B = gth(o1, idxB).reshape(HN, KP, TP)
    MA2, P1A2, P2A2 = _edge(G2A, o1, Zc2, Wd2T, 256, 0)
    MB2, P1B2, P2B2 = _edge(G2B, o1, Zc2, Wd2T, 256, 1)
    o2 = _bn_last(MA2, MB2, P1A2, P1B2, P2A2, P2B2, g2, b2)

    out = _final(o0, o1, o2, Wf.T, gf, bf)
    return out.reshape(B, N, 512)


# speedup vs baseline: 1.3883x; 1.0015x over previous
import functools

import jax
import jax.numpy as jnp
from jax import lax
from jax.experimental import pallas as pl
from jax.experimental.pallas import tpu as pltpu
from jax.experimental.pallas import tpu_sc as plsc

B, N, K = 4, 1024, 20
KP = 24
BN = B * N
HN = BN // 2
HB = B // 2
NEDGE = BN * K
EPS = 1e-5
BLK = 256
TP = 128
NW = 32
PW = HN // NW
PB = 128
NBLK = HN // PB


def _knn_body(b0, xt_ref, pts_ref, out_ref):
    b = pl.program_id(0)
    xt = xt_ref[0]
    pi = pts_ref[0]
    x0, x1, x2 = xt[0:1, :], xt[1:2, :], xt[2:3, :]
    p0, p1, p2 = pi[:, 0:1], pi[:, 1:2], pi[:, 2:3]
    g = jnp.dot(pi.astype(jnp.bfloat16), xt.astype(jnp.bfloat16),
                preferred_element_type=jnp.float32)
    sqj = (x0 * x0 + x1 * x1) + x2 * x2
    sqi = (p0 * p0 + p1 * p1) + p2 * p2
    d = (sqi + sqj) - 2.0 * g
    iota = lax.broadcasted_iota(jnp.int32, (BLK, N), 1)
    cols = []
    for _ in range(K):
        m = jnp.min(d, axis=1, keepdims=True)
        cand = jnp.where(d <= m, iota, N)
        amin = jnp.min(cand, axis=1, keepdims=True)
        cols.append(amin)
        d = jnp.where(iota == amin, jnp.float32(jnp.inf), d)
    cols += [cols[0]] * (KP - K)
    out_ref[...] = jnp.concatenate(cols, axis=1) + (b0 + b) * N


def _knn(x, b0):
    pts = jnp.transpose(x, (0, 2, 1))
    return pl.pallas_call(
        functools.partial(_knn_body, b0),
        grid=(HB, N // BLK),
        in_specs=[
            pl.BlockSpec((1, 3, N), lambda b, i: (b0 + b, 0, 0)),
            pl.BlockSpec((1, BLK, 3), lambda b, i: (b0 + b, i, 0)),
        ],
        out_specs=pl.BlockSpec((BLK, KP), lambda b, i: (b * (N // BLK) + i, 0)),
        out_shape=jax.ShapeDtypeStruct((HN, KP), jnp.int32),
    )(x, pts)


CH = 4
NCH = PW // CH
CR = CH * KP


def _make_gather():
    mesh = plsc.VectorSubcoreMesh(core_axis_name="c", subcore_axis_name="s")

    @functools.partial(
        pl.kernel,
        mesh=mesh,
        out_type=jax.ShapeDtypeStruct((HN * KP, TP), jnp.float32),
        scratch_types=[
            pltpu.VMEM((PW * KP,), jnp.int32),
            pltpu.VMEM((CR, TP), jnp.float32),
            pltpu.VMEM((CR, TP), jnp.float32),
            pltpu.VMEM((CR, TP), jnp.float32),
            pltpu.VMEM((CR, TP), jnp.float32),
            pltpu.SemaphoreType.DMA,
            pltpu.SemaphoreType.DMA,
            pltpu.SemaphoreType.DMA,
            pltpu.SemaphoreType.DMA,
            pltpu.SemaphoreType.DMA,
            pltpu.SemaphoreType.DMA,
            pltpu.SemaphoreType.DMA,
            pltpu.SemaphoreType.DMA,
        ],
    )
    def gather(x_hbm, idx_hbm, g_hbm, idx_v, r0, r1, r2, r3,
               sg0, sg1, sg2, sg3, sw0, sw1, sw2, sw3):
        wid = lax.axis_index("s") * 2 + lax.axis_index("c")
        base = wid * PW
        pltpu.sync_copy(idx_hbm.at[pl.ds(base * KP, PW * KP)], idx_v)

        rows = [r0, r1, r2, r3]
        sg = [sg0, sg1, sg2, sg3]
        sw = [sw0, sw1, sw2, sw3]

        def isl(c):
            return idx_v.at[pl.ds(c * CR, CR)]

        def gsl(c):
            return g_hbm.at[pl.ds(base * KP + c * CR, CR)]

        for j in range(NCH):
            b = j % 4
            if j >= 4:
                pltpu.make_async_copy(rows[b], gsl(j - 4), sw[b]).wait()
            pltpu.async_copy(x_hbm.at[isl(j)], rows[b], sg[b])
            if j >= 3:
                c = j - 3
                bc = c % 4
                pltpu.make_async_copy(x_hbm.at[isl(c)], rows[bc], sg[bc]).wait()
                pltpu.async_copy(rows[bc], gsl(c), sw[bc])
        for c in range(NCH - 3, NCH):
            bc = c % 4
            pltpu.make_async_copy(x_hbm.at[isl(c)], rows[bc], sg[bc]).wait()
            pltpu.async_copy(rows[bc], gsl(c), sw[bc])
        for c in range(NCH - 4, NCH):
            bc = c % 4
            pltpu.make_async_copy(rows[bc], gsl(c), sw[bc]).wait()

    return gather


def _edge_body(g_ref, x_ref, zc_ref, wd_ref, m_ref, p1_ref, p2_ref):
    X = x_ref[...]
    Zc = zc_ref[...]
    Wd = wd_ref[...].astype(jnp.bfloat16)
    h = jnp.dot((g_ref[:, 0, :] - X).astype(jnp.bfloat16), Wd,
                preferred_element_type=jnp.float32) + Zc
    M = h
    s1 = h
    s2 = h * h
    for k in range(1, K):
        h = jnp.dot((g_ref[:, k, :] - X).astype(jnp.bfloat16), Wd,
                    preferred_element_type=jnp.float32) + Zc
        M = jnp.maximum(M, h)
        s1 = s1 + h
        s2 = s2 + h * h
    m_ref[...] = M
    p1_ref[...] = jnp.sum(s1, axis=0, keepdims=True)[None]
    p2_ref[...] = jnp.sum(s2, axis=0, keepdims=True)[None]


def _edge(G3, table, Zc, WdT, Cn, half):
    off = half * NBLK
    return pl.pallas_call(
        _edge_body,
        grid=(NBLK,),
        in_specs=[
            pl.BlockSpec((PB, KP, TP), lambda i: (i, 0, 0)),
            pl.BlockSpec((PB, TP), lambda i: (off + i, 0)),
            pl.BlockSpec((PB, Cn), lambda i: (off + i, 0)),
            pl.BlockSpec((TP, Cn), lambda i: (0, 0)),
        ],
        out_specs=[
            pl.BlockSpec((PB, Cn), lambda i: (i, 0)),
            pl.BlockSpec((1, 1, Cn), lambda i: (i, 0, 0)),
            pl.BlockSpec((1, 1, Cn), lambda i: (i, 0, 0)),
        ],
        out_shape=[
            jax.ShapeDtypeStruct((HN, Cn), jnp.float32),
            jax.ShapeDtypeStruct((NBLK, 1, Cn), jnp.float32),
            jax.ShapeDtypeStruct((NBLK, 1, Cn), jnp.float32),
        ],
    )(G3, table, Zc, WdT)


def _bn_stats(p1a, p1b, p2a, p2b, g, b):
    s1 = (jnp.sum(p1a.reshape(NBLK, -1), axis=0, keepdims=True)
          + jnp.sum(p1b.reshape(NBLK, -1), axis=0, keepdims=True))
    s2 = (jnp.sum(p2a.reshape(NBLK, -1), axis=0, keepdims=True)
          + jnp.sum(p2b.reshape(NBLK, -1), axis=0, keepdims=True))
    mean = s1 * (1.0 / NEDGE)
    var = s2 * (1.0 / NEDGE) - mean * mean
    scale = g / jnp.sqrt(var + EPS)
    shift = b - mean * scale
    return scale, shift


def _bn_mid_body(ma_ref, mb_ref, p1a_ref, p1b_ref, p2a_ref, p2b_ref,
                 g_ref, b_ref, wc_ref, o_ref, zc_ref):
    scale, shift = _bn_stats(p1a_ref[...], p1b_ref[...],
                             p2a_ref[...], p2b_ref[...],
                             g_ref[...], b_ref[...])
    M = jnp.concatenate([ma_ref[...], mb_ref[...]], axis=0)
    o = jnp.maximum(M * scale + shift, 0.0)
    o_ref[...] = o
    zc_ref[...] = jnp.dot(o.astype(jnp.bfloat16),
                          wc_ref[...].astype(jnp.bfloat16),
                          preferred_element_type=jnp.float32)


def _bn_mid_pad_body(ma_ref, mb_ref, p1a_ref, p1b_ref, p2a_ref, p2b_ref,
                     g_ref, b_ref, wc_ref, o_ref, zc_ref, tab_ref):
    scale, shift = _bn_stats(p1a_ref[...], p1b_ref[...],
                             p2a_ref[...], p2b_ref[...],
                             g_ref[...], b_ref[...])
    M = jnp.concatenate([ma_ref[...], mb_ref[...]], axis=0)
    o = jnp.maximum(M * scale + shift, 0.0)
    o_ref[...] = o
    zc_ref[...] = jnp.dot(o.astype(jnp.bfloat16),
                          wc_ref[...].astype(jnp.bfloat16),
                          preferred_element_type=jnp.float32)
    tab_ref[...] = jnp.concatenate(
        [o, jnp.zeros((BN, TP - o.shape[1]), jnp.float32)], axis=1)


def _bn_last_body(ma_ref, mb_ref, p1a_ref, p1b_ref, p2a_ref, p2b_ref,
                  g_ref, b_ref, o_ref):
    scale, shift = _bn_stats(p1a_ref[...], p1b_ref[...],
                             p2a_ref[...], p2b_ref[...],
                             g_ref[...], b_ref[...])
    M = jnp.concatenate([ma_ref[...], mb_ref[...]], axis=0)
    o_ref[...] = jnp.maximum(M * scale + shift, 0.0)


def _bn_mid(MA, MB, P1A, P1B, P2A, P2B, g, b, WcT, Cn2, pad_table):
    C = MA.shape[1]
    outs = [jax.ShapeDtypeStruct((BN, C), jnp.float32),
            jax.ShapeDtypeStruct((BN, Cn2), jnp.float32)]
    body = _bn_mid_body
    if pad_table:
        outs.append(jax.ShapeDtypeStruct((BN, TP), jnp.float32))
        body = _bn_mid_pad_body
    return pl.pallas_call(
        body,
        out_shape=outs,
    )(MA, MB, P1A, P1B, P2A, P2B, g.reshape(1, C), b.reshape(1, C), WcT)


def _bn_last(MA, MB, P1A, P1B, P2A, P2B, g, b):
    C = MA.shape[1]
    return pl.pallas_call(
        _bn_last_body,
        out_shape=jax.ShapeDtypeStruct((BN, C), jnp.float32),
    )(MA, MB, P1A, P1B, P2A, P2B, g.reshape(1, C), b.reshape(1, C))


def _prep_body(x_ref, w_ref, zc_ref):
    zc_ref[...] = jnp.dot(x_ref[...].astype(jnp.bfloat16),
                          w_ref[...].astype(jnp.bfloat16),
                          preferred_element_type=jnp.float32)


def _prep(X0p, Wc0T):
    return pl.pallas_call(
        _prep_body,
        out_shape=jax.ShapeDtypeStruct((BN, 64), jnp.float32),
    )(X0p, Wc0T)


def _final_body(o0_ref, o1_ref, o2_ref, w0_ref, w1_ref, w2_ref, g_ref, b_ref,
                out_ref):
    h = (jnp.dot(o0_ref[...].astype(jnp.bfloat16),
                 w0_ref[...].astype(jnp.bfloat16),
                 preferred_element_type=jnp.float32)
         + jnp.dot(o1_ref[...].astype(jnp.bfloat16),
                   w1_ref[...].astype(jnp.bfloat16),
                   preferred_element_type=jnp.float32)
         + jnp.dot(o2_ref[...].astype(jnp.bfloat16),
                   w2_ref[...].astype(jnp.bfloat16),
                   preferred_element_type=jnp.float32))
    mean = jnp.mean(h, axis=0, keepdims=True)
    var = jnp.mean((h - mean) * (h - mean), axis=0, keepdims=True)
    scale = g_ref[...] / jnp.sqrt(var + EPS)
    shift = b_ref[...] - mean * scale
    out_ref[...] = jnp.maximum(h * scale + shift, 0.0)


def _final(o0, o1, o2, WfT, gf, bf):
    return pl.pallas_call(
        _final_body,
        out_shape=jax.ShapeDtypeStruct((BN, 512), jnp.float32),
    )(o0, o1, o2, WfT[:64], WfT[64:192], WfT[192:448],
      gf.reshape(1, 512), bf.reshape(1, 512))


def _pad_rows(W, rows):
    return jnp.pad(W, ((0, rows - W.shape[0]), (0, 0)))


def kernel(x, W0, g0, b0, W1, g1, b1, W2, g2, b2, Wf, gf, bf):
    idxA = _knn(x, 0).reshape(HN * KP)
    idxB = _knn(x, HB).reshape(HN * KP)

    Xr = jnp.transpose(x, (0, 2, 1)).reshape(BN, 3)
    X0p = jnp.pad(Xr, ((0, 0), (0, TP - 3)))

    gth = _make_gather()

    Wd0T = _pad_rows(W0[:, :3].T, TP)
    Wc0T = _pad_rows(W0[:, 3:].T, TP)
    Zc0 = _prep(X0p, Wc0T)
    G0A = gth(X0p, idxA).reshape(HN, KP, TP)
    G0B = gth(X0p, idxB).reshape(HN, KP, TP)
    MA0, P1A0, P2A0 = _edge(G0A, X0p, Zc0, Wd0T, 64, 0)
    MB0, P1B0, P2B0 = _edge(G0B, X0p, Zc0, Wd0T, 64, 1)
    o0, Zc1, tab1 = _bn_mid(MA0, MB0, P1A0, P1B0, P2A0, P2B0, g0, b0,
                            W1[:, 64:].T, 128, pad_table=True)

    Wd1T = _pad_rows(W1[:, :64].T, TP)
    G1A = gth(tab1, idxA).reshape(HN, KP, TP)
    G1B = gth(tab1, idxB).reshape(HN, KP, TP)
    MA1, P1A1, P2A1 = _edge(G1A, tab1, Zc1, Wd1T, 128, 0)
    MB1, P1B1, P2B1 = _edge(G1B, tab1, Zc1, Wd1T, 128, 1)
    o1, Zc2 = _bn_mid(MA1, MB1, P1A1, P1B1, P2A1, P2B1, g1, b1,
                      W2[:, 128:].T, 256, pad_table=False)

    Wd2T = W2[:, :128].T
    G2A = gth(o1, idxA).reshape(HN, KP, TP)
    G2B = gth(o1, idxB).reshape(HN, KP, TP)
    MA2, P1A2, P2A2 = _edge(G2A, o1, Zc2, Wd2T, 256, 0)
    MB2, P1B2, P2B2 = _edge(G2B, o1, Zc2, Wd2T, 256, 1)
    o2 = _bn_last(MA2, MB2, P1A2, P1B2, P2A2, P2B2, g2, b2)

    out = _final(o0, o1, o2, Wf.T, gf, bf)
    return out.reshape(B, N, 512)
